# Initial kernel scaffold; baseline (speedup 1.0000x reference)
import jax
import jax.numpy as jnp
from jax.experimental import pallas as pl


def kernel(feat_l0, feat_l1, feat_l2, feat_l3, feat_l4, x, cls_w0, cls_b0, reg_w0, reg_b0, cls_w1, cls_b1, reg_w1, reg_b1, cls_w2, cls_b2, reg_w2, reg_b2, cls_w3, cls_b3, reg_w3, reg_b3, cls_hw, cls_hb, reg_hw, reg_hb, anc_l0, anc_l1, anc_l2, anc_l3, anc_l4):
    raise NotImplementedError("write your pallas kernel here")


# speedup vs baseline: 2.5615x; 2.5615x over previous
import functools

import jax
import jax.numpy as jnp
import numpy as np
from jax.experimental import pallas as pl
from jax.experimental.pallas import tpu as pltpu

STRIDES = (8, 16, 32, 64, 128)
IMG = 512
CH = 256
NC = 80
A = 9
STACKED = 4
NMS_PRE = 1000
SCORE_THR = 0.05
IOU_THR = 0.5
MAX_PER_IMG = 100
MAX_RATIO = float(np.abs(np.log(16.0 / 1000.0)))


def _nms_kernel(cbo_ref, cbot_ref, cs_ref, ks_ref, iou_ref):
    K = cbo_ref.shape[0]
    x1 = cbo_ref[:, 0:1]
    y1 = cbo_ref[:, 1:2]
    x2 = cbo_ref[:, 2:3]
    y2 = cbo_ref[:, 3:4]
    x1t = cbot_ref[0:1, :]
    y1t = cbot_ref[1:2, :]
    x2t = cbot_ref[2:3, :]
    y2t = cbot_ref[3:4, :]

    area = jnp.clip(x2 - x1, 0.0) * jnp.clip(y2 - y1, 0.0)
    areat = jnp.clip(x2t - x1t, 0.0) * jnp.clip(y2t - y1t, 0.0)
    ix1 = jnp.maximum(x1, x1t)
    iy1 = jnp.maximum(y1, y1t)
    ix2 = jnp.minimum(x2, x2t)
    iy2 = jnp.minimum(y2, y2t)
    inter = jnp.clip(ix2 - ix1, 0.0) * jnp.clip(iy2 - iy1, 0.0)
    union = area + areat - inter
    iou_ref[...] = inter / jnp.maximum(union, 1e-6)

    cs = cs_ref[...]
    idx = jax.lax.broadcasted_iota(jnp.int32, (1, K), 1)
    keep0 = (cs > 0.0).astype(jnp.float32)

    def block(j, keep):
        rows = iou_ref[pl.ds(j * 8, 8), :]
        for r in range(8):
            i = j * 8 + r
            srow = (rows[r:r + 1, :] > IOU_THR).astype(jnp.float32)
            later = (idx > i).astype(jnp.float32)
            keep_i = jnp.sum(keep * (idx == i).astype(jnp.float32),
                             axis=1, keepdims=True)
            sup = srow * later * keep_i
            keep = keep * (1.0 - sup)
        return keep

    keep = jax.lax.fori_loop(0, K // 8, block, keep0)
    ks_ref[...] = cs * keep


def _nms_pallas(cbo, cs):
    K = cbo.shape[0]
    ks = pl.pallas_call(
        _nms_kernel,
        out_shape=jax.ShapeDtypeStruct((1, K), jnp.float32),
        scratch_shapes=[pltpu.VMEM((K, K), jnp.float32)],
    )(cbo, jnp.transpose(cbo), cs[None, :])
    return ks[0]


def _conv(x, w, b):
    y = jax.lax.conv_general_dilated(
        x, w, (1, 1), "SAME", dimension_numbers=("NCHW", "OIHW", "NCHW"))
    return y + b[None, :, None, None]


def _level(feat, p, anchors, img_hw):
    c = feat
    r = feat
    for t in range(STACKED):
        c = jax.nn.relu(_conv(c, p["cls_w%d" % t], p["cls_b%d" % t]))
        r = jax.nn.relu(_conv(r, p["reg_w%d" % t], p["reg_b%d" % t]))
    cls = _conv(c, p["cls_hw"], p["cls_hb"])
    reg = _conv(r, p["reg_hw"], p["reg_hb"])
    Bn = cls.shape[0]
    H = cls.shape[2]
    W = cls.shape[3]
    scores = jax.nn.sigmoid(
        jnp.transpose(cls, (0, 2, 3, 1)).reshape(Bn, H * W * A, NC))
    deltas = jnp.transpose(reg, (0, 2, 3, 1)).reshape(Bn, H * W * A, 4)
    anc = anchors[None]
    pw = anc[..., 2] - anc[..., 0]
    ph = anc[..., 3] - anc[..., 1]
    px = (anc[..., 0] + anc[..., 2]) * 0.5
    py = (anc[..., 1] + anc[..., 3]) * 0.5
    dx = deltas[..., 0]
    dy = deltas[..., 1]
    dw = jnp.clip(deltas[..., 2], -MAX_RATIO, MAX_RATIO)
    dh = jnp.clip(deltas[..., 3], -MAX_RATIO, MAX_RATIO)
    gw = pw * jnp.exp(dw)
    gh = ph * jnp.exp(dh)
    gx = px + pw * dx
    gy = py + ph * dy
    x1 = jnp.clip(gx - gw * 0.5, 0.0, img_hw[1])
    y1 = jnp.clip(gy - gh * 0.5, 0.0, img_hw[0])
    x2 = jnp.clip(gx + gw * 0.5, 0.0, img_hw[1])
    y2 = jnp.clip(gy + gh * 0.5, 0.0, img_hw[0])
    boxes = jnp.stack([x1, y1, x2, y2], axis=-1)
    if scores.shape[1] > NMS_PRE:
        ms = jnp.max(scores, axis=2)
        _, inds = jax.lax.top_k(ms, NMS_PRE)
        scores = jnp.take_along_axis(scores, inds[..., None], axis=1)
        boxes = jnp.take_along_axis(boxes, inds[..., None], axis=1)
    return scores, boxes


def _image_nms(scores, boxes):
    flat = jnp.where(scores > SCORE_THR, scores, 0.0).reshape(-1)
    K = min(NMS_PRE, flat.shape[0])
    cs, ci = jax.lax.top_k(flat, K)
    bi = ci // NC
    cls = ci % NC
    cb = boxes[bi]
    off = cls.astype(jnp.float32) * (2.0 * IMG)
    ks = _nms_pallas(cb + off[:, None], cs)
    fs, fi = jax.lax.top_k(ks, MAX_PER_IMG)
    fb = cb[fi]
    fc = cls[fi]
    nd = jnp.sum((fs > 0.0).astype(jnp.int32))
    return nd, fb, fs, fc


def kernel(feat_l0, feat_l1, feat_l2, feat_l3, feat_l4, x,
           cls_w0, cls_b0, reg_w0, reg_b0,
           cls_w1, cls_b1, reg_w1, reg_b1,
           cls_w2, cls_b2, reg_w2, reg_b2,
           cls_w3, cls_b3, reg_w3, reg_b3,
           cls_hw, cls_hb, reg_hw, reg_hb,
           anc_l0, anc_l1, anc_l2, anc_l3, anc_l4):
    inp = dict(locals())
    img_hw = (x.shape[2], x.shape[3])
    all_s = []
    all_b = []
    for l in range(len(STRIDES)):
        s, b = _level(inp["feat_l%d" % l], inp, inp["anc_l%d" % l], img_hw)
        all_s.append(s)
        all_b.append(b)
    scores = jnp.concatenate(all_s, axis=1)
    boxes = jnp.concatenate(all_b, axis=1)
    ms = jnp.max(scores, axis=2)
    topk_pre = min(max(1000, NMS_PRE), scores.shape[1])
    _, inds = jax.lax.top_k(ms, topk_pre)
    scores = jnp.take_along_axis(scores, inds[..., None], axis=1)
    boxes = jnp.take_along_axis(boxes, inds[..., None], axis=1)
    return jax.vmap(_image_nms)(scores, boxes)
